# baseline (speedup 1.0000x reference)
import functools

import jax
import jax.numpy as jnp
from jax import lax
from jax.experimental import pallas as pl
from jax.experimental.pallas import tpu as pltpu
from jax.experimental.pallas import tpu_sc as plsc

N = 10000
E = 160000
F = 256
HALF = 128
G = 64
C = 64
NP = 10240
EPAD = 163840
NSUB = 16
NCORE = 2
K = 128
EPT = EPAD // NSUB
NBLK = EPT // K
RPT = NP // NSUB
GRP = 8
NGRP = NBLK // GRP
DEGW = 128
R = 512
NG = NP // R


def _sc_agg_body(table, src2, dst, zrows, agg_out,
                 src_all, dgrp, r0, r1, agg_sh, g0, g1, s0, s1):
    c = lax.axis_index("c")
    s = lax.axis_index("s")
    rows = [r0, r1]
    gsems = [g0, g1]
    ssems = [s0, s1]
    pltpu.sync_copy(zrows, r0)
    for m in range(RPT // K):
        pltpu.sync_copy(r0, agg_sh.at[pl.ds(s * RPT + m * K, K)])
    pltpu.sync_copy(src2.at[c * NSUB + s], src_all)
    plsc.subcore_barrier()

    def super_it(g, carry):
        gh = [pltpu.async_copy(
                  table.at[src_all.at[pl.ds((g * GRP + b) * K, K)]],
                  rows[b], gsems[b]) for b in range(2)]
        pltpu.sync_copy(dst.at[s, g], dgrp)
        for qq in range(GRP // 2):
            sh = []
            for b in range(2):
                gh[b].wait()
                sh.append(pltpu.async_copy(
                    rows[b], agg_sh.at[dgrp.at[2 * qq + b]],
                    ssems[b], add=True))
            for b in range(2):
                sh[b].wait()
            if qq < GRP // 2 - 1:
                gh = [pltpu.async_copy(
                          table.at[src_all.at[
                              pl.ds((g * GRP + 2 * (qq + 1) + b) * K, K)]],
                          rows[b], gsems[b]) for b in range(2)]
        return carry

    lax.fori_loop(0, NGRP, super_it, 0)
    plsc.subcore_barrier()
    for m in range(RPT // K):
        pltpu.sync_copy(agg_sh.at[pl.ds(s * RPT + m * K, K)], r0)
        pltpu.sync_copy(r0, agg_out.at[pl.ds(c * NP + s * RPT + m * K, K)])


def _make_sc_agg():
    return pl.kernel(
        _sc_agg_body,
        mesh=plsc.VectorSubcoreMesh(core_axis_name="c", subcore_axis_name="s"),
        out_type=jax.ShapeDtypeStruct((NCORE * NP, HALF), jnp.float32),
        scratch_types=[
            pltpu.VMEM((EPT + 2 * K,), jnp.int32),
            pltpu.VMEM((GRP, K), jnp.int32),
            pltpu.VMEM((K, HALF), jnp.float32),
            pltpu.VMEM((K, HALF), jnp.float32),
            pltpu.VMEM_SHARED((NP, HALF), jnp.float32),
            pltpu.SemaphoreType.DMA,
            pltpu.SemaphoreType.DMA,
            pltpu.SemaphoreType.DMA,
            pltpu.SemaphoreType.DMA,
        ],
    )


def _sc_deg_body(dst, zdeg, ones, deg_out, dst_all, onesv, dbuf, deg_sh, sem):
    c = lax.axis_index("c")
    s = lax.axis_index("s")
    pltpu.sync_copy(zdeg, dbuf)
    for m in range(RPT // K):
        pltpu.sync_copy(dbuf, deg_sh.at[pl.ds(s * RPT + m * K, K)])
    pltpu.sync_copy(ones, onesv)
    pltpu.sync_copy(dst.at[s], dst_all)
    plsc.subcore_barrier()

    def body(j, carry):
        pltpu.sync_copy(onesv, deg_sh.at[dst_all.at[j]], add=True)
        return carry

    lax.fori_loop(0, NBLK // NCORE, lambda j, car: body(j * NCORE + c, car), 0)
    plsc.subcore_barrier()
    for m in range(RPT // K):
        pltpu.sync_copy(deg_sh.at[pl.ds(s * RPT + m * K, K)], dbuf)
        pltpu.sync_copy(dbuf, deg_out.at[pl.ds(c * NP + s * RPT + m * K, K)])


def _make_sc_deg():
    return pl.kernel(
        _sc_deg_body,
        mesh=plsc.VectorSubcoreMesh(core_axis_name="c", subcore_axis_name="s"),
        out_type=jax.ShapeDtypeStruct((NCORE * NP, DEGW), jnp.float32),
        scratch_types=[
            pltpu.VMEM((NBLK, K), jnp.int32),
            pltpu.VMEM((K, DEGW), jnp.float32),
            pltpu.VMEM((K, DEGW), jnp.float32),
            pltpu.VMEM_SHARED((NP, DEGW), jnp.float32),
            pltpu.SemaphoreType.DMA,
        ],
    )


def _sage_tc_body(h_ref, a_ref, d_ref, ws_ref, wn_ref, b_ref, o_ref):
    r = 1.0 / jnp.maximum(d_ref[0, :, 0:1] + d_ref[1, :, 0:1], 1.0)
    o = (jnp.dot(h_ref[0], ws_ref[0:HALF, :], preferred_element_type=jnp.float32)
         + jnp.dot(h_ref[1], ws_ref[HALF:F, :], preferred_element_type=jnp.float32)
         + jnp.dot(a_ref[0] * r, wn_ref[0:HALF, :], preferred_element_type=jnp.float32)
         + jnp.dot(a_ref[1] * r, wn_ref[HALF:F, :], preferred_element_type=jnp.float32)
         + b_ref[...])
    o = jnp.maximum(o, 0.0)
    o_ref[0] = o[:, 0:HALF]
    o_ref[1] = o[:, HALF:F]


def _sage_tc(h2, agg2, deg, Ws, Wn, b):
    return pl.pallas_call(
        _sage_tc_body,
        grid=(NG,),
        in_specs=[
            pl.BlockSpec((2, R, HALF), lambda i: (0, i, 0)),
            pl.BlockSpec((2, R, HALF), lambda i: (0, i, 0)),
            pl.BlockSpec((2, R, DEGW), lambda i: (0, i, 0)),
            pl.BlockSpec((F, F), lambda i: (0, 0)),
            pl.BlockSpec((F, F), lambda i: (0, 0)),
            pl.BlockSpec((1, F), lambda i: (0, 0)),
        ],
        out_specs=pl.BlockSpec((2, R, HALF), lambda i: (0, i, 0)),
        out_shape=jax.ShapeDtypeStruct((2, NP, HALF), jnp.float32),
    )(h2, agg2, deg, Ws, Wn, b)


def _pool_body(b_ref, h_ref, w1_ref, b1_ref, w2_ref, b2_ref, o_ref, acc, cnt):
    i = pl.program_id(0)

    @pl.when(i == 0)
    def _():
        acc[...] = jnp.zeros_like(acc)
        cnt[...] = jnp.zeros_like(cnt)

    hcat = jnp.concatenate([h_ref[0], h_ref[1]], axis=1)
    bid = b_ref[0]
    oh = (lax.broadcasted_iota(jnp.int32, (G, R), 0) == bid).astype(jnp.float32)
    acc[...] += jnp.dot(oh, hcat, preferred_element_type=jnp.float32)
    cnt[...] = cnt[...] + jnp.sum(oh, axis=1, keepdims=True)

    @pl.when(i == NG - 1)
    def _():
        pooled = acc[...] / jnp.maximum(cnt[...][:, 0:1], 1.0)
        hmid = jnp.maximum(
            jnp.dot(pooled, w1_ref[...], preferred_element_type=jnp.float32)
            + b1_ref[...], 0.0)
        o_ref[...] = (jnp.dot(hmid, w2_ref[...], preferred_element_type=jnp.float32)
                      + b2_ref[...])


def _pool_tc(batch3, h2, W1, b1, W2, b2):
    return pl.pallas_call(
        _pool_body,
        grid=(NG,),
        in_specs=[
            pl.BlockSpec((1, 1, R), lambda i: (i, 0, 0)),
            pl.BlockSpec((2, R, HALF), lambda i: (0, i, 0)),
            pl.BlockSpec((F, F), lambda i: (0, 0)),
            pl.BlockSpec((1, F), lambda i: (0, 0)),
            pl.BlockSpec((F, C), lambda i: (0, 0)),
            pl.BlockSpec((1, C), lambda i: (0, 0)),
        ],
        out_specs=pl.BlockSpec((G, C), lambda i: (0, 0)),
        out_shape=jax.ShapeDtypeStruct((G, C), jnp.float32),
        scratch_shapes=[
            pltpu.VMEM((G, F), jnp.float32),
            pltpu.VMEM((G, HALF), jnp.float32),
        ],
    )(batch3, h2, W1, b1, W2, b2)


def kernel(x, edge_index, batch, W1_self, b1_self, W1_neigh, b1_neigh,
           W2_self, b2_self, W2_neigh, b2_neigh, W3_self, b3_self, W3_neigh,
           b3_neigh, W_lin1, b_lin1, W_lin2, b_lin2):
    f32 = jnp.float32
    src = edge_index[0]
    dst = edge_index[1]

    xp = jnp.pad(x, ((0, NP - N), (0, 0)))
    h = jnp.stack([xp[:, 0:HALF], xp[:, HALF:F]])
    srcp = jnp.pad(src, (0, EPAD - E))
    dstp = jnp.pad(dst, (0, EPAD - E), constant_values=N)
    src2 = jnp.pad(
        jnp.concatenate([srcp, srcp + NP]).reshape(2 * NSUB, EPT),
        ((0, 0), (0, 2 * K)))
    dst3 = dstp.reshape(NSUB, NBLK, K)
    dst4 = dstp.reshape(NSUB, NGRP, GRP, K)
    zrows = jnp.zeros((K, HALF), f32)
    zdeg = jnp.zeros((K, DEGW), f32)
    ones = jnp.ones((K, DEGW), f32)
    batch3 = jnp.pad(batch, (0, NP - N), constant_values=G).reshape(NG, 1, R)

    sc_agg = _make_sc_agg()
    deg = _make_sc_deg()(dst3, zdeg, ones).reshape(2, NP, DEGW)
    layers = [
        (W1_self, b1_self, W1_neigh, b1_neigh),
        (W2_self, b2_self, W2_neigh, b2_neigh),
        (W3_self, b3_self, W3_neigh, b3_neigh),
    ]
    for Ws, bs, Wn, bn in layers:
        table = h.reshape(NCORE * NP, HALF)
        aggf = sc_agg(table, src2, dst4, zrows)
        h = _sage_tc(h, aggf.reshape(2, NP, HALF), deg, Ws, Wn,
                     (bs + bn).reshape(1, F))

    return _pool_tc(batch3, h, W_lin1, b_lin1.reshape(1, F),
                    W_lin2, b_lin2.reshape(1, C))


# speedup vs baseline: 1.0026x; 1.0026x over previous
import functools

import jax
import jax.numpy as jnp
from jax import lax
from jax.experimental import pallas as pl
from jax.experimental.pallas import tpu as pltpu
from jax.experimental.pallas import tpu_sc as plsc

N = 10000
E = 160000
F = 256
HALF = 128
G = 64
C = 64
NP = 10240
EPAD = 163840
NSUB = 16
NCORE = 2
K = 128
EPT = EPAD // NSUB
NBLK = EPT // K
RPT = NP // NSUB
GRP = 2
NGRP = NBLK // GRP
DEGW = 128
R = 512
NG = NP // R


def _sc_agg_body(table, src2, dst, zrows, agg_out,
                 src_all, dgrp, r0, r1, agg_sh, g0, g1, s0, s1):
    c = lax.axis_index("c")
    s = lax.axis_index("s")
    rows = [r0, r1]
    gsems = [g0, g1]
    ssems = [s0, s1]
    pltpu.sync_copy(zrows, r0)
    for m in range(RPT // K):
        pltpu.sync_copy(r0, agg_sh.at[pl.ds(s * RPT + m * K, K)])
    pltpu.sync_copy(src2.at[c * NSUB + s], src_all)
    plsc.subcore_barrier()

    def quad(q, carry):
        gh = [pltpu.async_copy(
                  table.at[src_all.at[pl.ds((q * GRP + b) * K, K)]],
                  rows[b], gsems[b]) for b in range(GRP)]
        pltpu.sync_copy(dst.at[s, q], dgrp)
        sh = []
        for b in range(GRP):
            gh[b].wait()
            sh.append(pltpu.async_copy(rows[b], agg_sh.at[dgrp.at[b]],
                                       ssems[b], add=True))
        for b in range(GRP):
            sh[b].wait()
        return carry

    lax.fori_loop(0, NGRP, quad, 0)
    plsc.subcore_barrier()
    for m in range(RPT // K):
        pltpu.sync_copy(agg_sh.at[pl.ds(s * RPT + m * K, K)], r0)
        pltpu.sync_copy(r0, agg_out.at[pl.ds(c * NP + s * RPT + m * K, K)])


def _make_sc_agg():
    return pl.kernel(
        _sc_agg_body,
        mesh=plsc.VectorSubcoreMesh(core_axis_name="c", subcore_axis_name="s"),
        out_type=jax.ShapeDtypeStruct((NCORE * NP, HALF), jnp.float32),
        scratch_types=[
            pltpu.VMEM((EPT + 2 * K,), jnp.int32),
            pltpu.VMEM((GRP, K), jnp.int32),
            pltpu.VMEM((K, HALF), jnp.float32),
            pltpu.VMEM((K, HALF), jnp.float32),
            pltpu.VMEM_SHARED((NP, HALF), jnp.float32),
            pltpu.SemaphoreType.DMA,
            pltpu.SemaphoreType.DMA,
            pltpu.SemaphoreType.DMA,
            pltpu.SemaphoreType.DMA,
        ],
    )


def _sc_deg_body(dst, zdeg, ones, deg_out, dst_all, onesv, dbuf, deg_sh, sem):
    c = lax.axis_index("c")
    s = lax.axis_index("s")
    pltpu.sync_copy(zdeg, dbuf)
    for m in range(RPT // K):
        pltpu.sync_copy(dbuf, deg_sh.at[pl.ds(s * RPT + m * K, K)])
    pltpu.sync_copy(ones, onesv)
    pltpu.sync_copy(dst.at[s], dst_all)
    plsc.subcore_barrier()

    def body(j, carry):
        pltpu.sync_copy(onesv, deg_sh.at[dst_all.at[j]], add=True)
        return carry

    lax.fori_loop(0, NBLK // NCORE, lambda j, car: body(j * NCORE + c, car), 0)
    plsc.subcore_barrier()
    for m in range(RPT // K):
        pltpu.sync_copy(deg_sh.at[pl.ds(s * RPT + m * K, K)], dbuf)
        pltpu.sync_copy(dbuf, deg_out.at[pl.ds(c * NP + s * RPT + m * K, K)])


def _make_sc_deg():
    return pl.kernel(
        _sc_deg_body,
        mesh=plsc.VectorSubcoreMesh(core_axis_name="c", subcore_axis_name="s"),
        out_type=jax.ShapeDtypeStruct((NCORE * NP, DEGW), jnp.float32),
        scratch_types=[
            pltpu.VMEM((NBLK, K), jnp.int32),
            pltpu.VMEM((K, DEGW), jnp.float32),
            pltpu.VMEM((K, DEGW), jnp.float32),
            pltpu.VMEM_SHARED((NP, DEGW), jnp.float32),
            pltpu.SemaphoreType.DMA,
        ],
    )


def _sage_tc_body(h_ref, a_ref, d_ref, ws_ref, wn_ref, b_ref, o_ref):
    r = 1.0 / jnp.maximum(d_ref[0, :, 0:1] + d_ref[1, :, 0:1], 1.0)
    o = (jnp.dot(h_ref[0], ws_ref[0:HALF, :], preferred_element_type=jnp.float32)
         + jnp.dot(h_ref[1], ws_ref[HALF:F, :], preferred_element_type=jnp.float32)
         + jnp.dot(a_ref[0] * r, wn_ref[0:HALF, :], preferred_element_type=jnp.float32)
         + jnp.dot(a_ref[1] * r, wn_ref[HALF:F, :], preferred_element_type=jnp.float32)
         + b_ref[...])
    o = jnp.maximum(o, 0.0)
    o_ref[0] = o[:, 0:HALF]
    o_ref[1] = o[:, HALF:F]


def _sage_tc(h2, agg2, deg, Ws, Wn, b):
    return pl.pallas_call(
        _sage_tc_body,
        grid=(NG,),
        in_specs=[
            pl.BlockSpec((2, R, HALF), lambda i: (0, i, 0)),
            pl.BlockSpec((2, R, HALF), lambda i: (0, i, 0)),
            pl.BlockSpec((2, R, DEGW), lambda i: (0, i, 0)),
            pl.BlockSpec((F, F), lambda i: (0, 0)),
            pl.BlockSpec((F, F), lambda i: (0, 0)),
            pl.BlockSpec((1, F), lambda i: (0, 0)),
        ],
        out_specs=pl.BlockSpec((2, R, HALF), lambda i: (0, i, 0)),
        out_shape=jax.ShapeDtypeStruct((2, NP, HALF), jnp.float32),
    )(h2, agg2, deg, Ws, Wn, b)


def _pool_body(b_ref, h_ref, w1_ref, b1_ref, w2_ref, b2_ref, o_ref, acc, cnt):
    i = pl.program_id(0)

    @pl.when(i == 0)
    def _():
        acc[...] = jnp.zeros_like(acc)
        cnt[...] = jnp.zeros_like(cnt)

    hcat = jnp.concatenate([h_ref[0], h_ref[1]], axis=1)
    bid = b_ref[0]
    oh = (lax.broadcasted_iota(jnp.int32, (G, R), 0) == bid).astype(jnp.float32)
    acc[...] += jnp.dot(oh, hcat, preferred_element_type=jnp.float32)
    cnt[...] = cnt[...] + jnp.sum(oh, axis=1, keepdims=True)

    @pl.when(i == NG - 1)
    def _():
        pooled = acc[...] / jnp.maximum(cnt[...][:, 0:1], 1.0)
        hmid = jnp.maximum(
            jnp.dot(pooled, w1_ref[...], preferred_element_type=jnp.float32)
            + b1_ref[...], 0.0)
        o_ref[...] = (jnp.dot(hmid, w2_ref[...], preferred_element_type=jnp.float32)
                      + b2_ref[...])


def _pool_tc(batch3, h2, W1, b1, W2, b2):
    return pl.pallas_call(
        _pool_body,
        grid=(NG,),
        in_specs=[
            pl.BlockSpec((1, 1, R), lambda i: (i, 0, 0)),
            pl.BlockSpec((2, R, HALF), lambda i: (0, i, 0)),
            pl.BlockSpec((F, F), lambda i: (0, 0)),
            pl.BlockSpec((1, F), lambda i: (0, 0)),
            pl.BlockSpec((F, C), lambda i: (0, 0)),
            pl.BlockSpec((1, C), lambda i: (0, 0)),
        ],
        out_specs=pl.BlockSpec((G, C), lambda i: (0, 0)),
        out_shape=jax.ShapeDtypeStruct((G, C), jnp.float32),
        scratch_shapes=[
            pltpu.VMEM((G, F), jnp.float32),
            pltpu.VMEM((G, HALF), jnp.float32),
        ],
    )(batch3, h2, W1, b1, W2, b2)


def kernel(x, edge_index, batch, W1_self, b1_self, W1_neigh, b1_neigh,
           W2_self, b2_self, W2_neigh, b2_neigh, W3_self, b3_self, W3_neigh,
           b3_neigh, W_lin1, b_lin1, W_lin2, b_lin2):
    f32 = jnp.float32
    src = edge_index[0]
    dst = edge_index[1]

    xp = jnp.pad(x, ((0, NP - N), (0, 0)))
    h = jnp.stack([xp[:, 0:HALF], xp[:, HALF:F]])
    srcp = jnp.pad(src, (0, EPAD - E))
    dstp = jnp.pad(dst, (0, EPAD - E), constant_values=N)
    src2 = jnp.pad(
        jnp.concatenate([srcp, srcp + NP]).reshape(2 * NSUB, EPT),
        ((0, 0), (0, 2 * K)))
    dst3 = dstp.reshape(NSUB, NBLK, K)
    dst4 = dstp.reshape(NSUB, NGRP, GRP, K)
    zrows = jnp.zeros((K, HALF), f32)
    zdeg = jnp.zeros((K, DEGW), f32)
    ones = jnp.ones((K, DEGW), f32)
    batch3 = jnp.pad(batch, (0, NP - N), constant_values=G).reshape(NG, 1, R)

    sc_agg = _make_sc_agg()
    deg = _make_sc_deg()(dst3, zdeg, ones).reshape(2, NP, DEGW)
    layers = [
        (W1_self, b1_self, W1_neigh, b1_neigh),
        (W2_self, b2_self, W2_neigh, b2_neigh),
        (W3_self, b3_self, W3_neigh, b3_neigh),
    ]
    for Ws, bs, Wn, bn in layers:
        table = h.reshape(NCORE * NP, HALF)
        aggf = sc_agg(table, src2, dst4, zrows)
        h = _sage_tc(h, aggf.reshape(2, NP, HALF), deg, Ws, Wn,
                     (bs + bn).reshape(1, F))

    return _pool_tc(batch3, h, W_lin1, b_lin1.reshape(1, F),
                    W_lin2, b_lin2.reshape(1, C))
